# Initial kernel scaffold; baseline (speedup 1.0000x reference)
import jax
import jax.numpy as jnp
from jax.experimental import pallas as pl


def kernel(tokens_B, output_A, output_C, t, Wq, bq, Wk, bk, Wv, bv, Wo, bo, ln_g, ln_b, Wt1, bt1, Wt2, bt2, Wg1, bg1, Wg2, bg2, expert_bias):
    raise NotImplementedError("write your pallas kernel here")


# speedup vs baseline: 1.6035x; 1.6035x over previous
import math

import jax
import jax.numpy as jnp
from jax import lax
from jax.experimental import pallas as pl
from jax.experimental.pallas import tpu as pltpu
from jax.experimental.pallas import tpu_sc as plsc

B, NB, NA, NC = 2, 2048, 512, 512
D, H, E, TOPK, TMAX = 768, 12, 16, 2, 1000
NK = NA + NC
DH = D // H
TAU_MIN, TAU_MAX = 0.5, 2.0
W_MIN, W_MAX = 0.1, 0.2
CAP_LOW, CAP_HIGH = 0.5, 0.6
FLOOR = min(0.05, 0.15 / E)
ALPHA = min(FLOOR * E, 1.0)
BIAS_COEFF = 0.01

BLKQ = 512
NQ = NB // BLKQ
NWORKERS = 32


_PREC = lax.Precision.DEFAULT


def _dot(a, b):
    return jnp.dot(a, b, preferred_element_type=jnp.float32, precision=_PREC)


def _ctx_body(ctx_ref, wk_ref, bk_ref, wv_ref, bv_ref, t_ref, freq_ref,
              wt1_ref, bt1_ref, wt2_ref, bt2_ref, eb_ref,
              k_ref, v_ref, tg_ref, pen_ref):
    ctx = ctx_ref[0]
    k_ref[0] = _dot(ctx, wk_ref[...]) + bk_ref[...]
    v_ref[0] = _dot(ctx, wv_ref[...]) + bv_ref[...]
    args = t_ref[0, pl.program_id(0)] * freq_ref[...]
    emb = jnp.concatenate([jnp.sin(args), jnp.cos(args)], axis=-1)
    h = _dot(emb, wt1_ref[...]) + bt1_ref[...]
    h = h * jax.nn.sigmoid(h)
    tg_ref[0] = _dot(h, wt2_ref[...]) + bt2_ref[...]
    pen_ref[...] = BIAS_COEFF * jnp.sum(eb_ref[...] ** 2, keepdims=True)


def _main_body(x_ref, k_ref, v_ref, wq_ref, bq_ref, wo_ref, bo_ref,
               lng_ref, lnb_ref, wg1_ref, tg_ref, wg2_ref,
               bg2e_ref, out_ref):
    x = x_ref[0]
    q = _dot(x, wq_ref[...]) + bq_ref[...]
    k = k_ref[0]
    v = v_ref[0]
    scale = 1.0 / math.sqrt(DH)
    heads = []
    for h in range(H):
        qh = q[:, h * DH:(h + 1) * DH]
        kh = k[:, h * DH:(h + 1) * DH]
        vh = v[:, h * DH:(h + 1) * DH]
        s = lax.dot_general(qh, kh, (((1,), (1,)), ((), ())),
                            preferred_element_type=jnp.float32,
                            precision=_PREC) * scale
        m = jnp.max(s, axis=-1, keepdims=True)
        e = jnp.exp(s - m)
        den = jnp.sum(e, axis=-1, keepdims=True)
        heads.append(_dot(e, vh) / den)
    attn = jnp.concatenate(heads, axis=-1)
    o = _dot(attn, wo_ref[...]) + bo_ref[...] + x
    mu = jnp.mean(o, axis=-1, keepdims=True)
    var = jnp.mean((o - mu) ** 2, axis=-1, keepdims=True)
    ctxb = (o - mu) / jnp.sqrt(var + 1e-5) * lng_ref[...] + lnb_ref[...]
    gate_in = jnp.concatenate(
        [x, ctxb, jnp.broadcast_to(tg_ref[0], (BLKQ, D))], axis=-1)
    hh = jnp.maximum(_dot(gate_in, wg1_ref[...]), 0.0)
    out_ref[0] = (_dot(hh, wg2_ref[...])
                  + bg2e_ref[...])


def _route_body(lg_hbm, tc_hbm, disp_hbm, comb_hbm, lbuf, tbuf, dbuf, cbuf):
    wid = lax.axis_index("c") * 16 + lax.axis_index("s")
    tok_w = (B * NB) // NWORKERS
    base = wid * tok_w
    pltpu.sync_copy(lg_hbm.at[pl.ds(base, tok_w)], lbuf)
    pltpu.sync_copy(tc_hbm.at[wid], tbuf)
    tn = tbuf[...]
    tau = TAU_MIN + (TAU_MAX - TAU_MIN) * tn
    w_sh = W_MIN + (W_MAX - W_MIN) * tn
    cap = CAP_LOW + (CAP_HIGH - CAP_LOW) * tn
    lane = lax.iota(jnp.int32, 16)
    is0 = lane == 0

    def step(i, carry):
        z = lbuf[i] / tau
        m = jnp.max(z)
        e = jnp.exp(z - m)
        p = e / jnp.sum(e)
        p = (1.0 - ALPHA) * p + (ALPHA / E)
        rest = jnp.where(is0, 0.0, p)
        rest = rest / jnp.maximum(jnp.sum(rest), 1e-8)
        probs = jnp.where(is0, w_sh, (1.0 - w_sh) * rest)
        capped = jnp.minimum(probs, cap)
        headroom = jnp.maximum(cap - capped, 0.0)
        hs = jnp.maximum(jnp.sum(headroom), 1e-8)
        excess = jnp.sum(probs - capped)
        capped = capped + excess * headroom / hs
        m1 = jnp.max(capped)
        i1 = jnp.min(jnp.where(capped == m1, lane, E))
        sel1 = lane == i1
        masked = jnp.where(sel1, -1e30, capped)
        m2 = jnp.max(masked)
        i2 = jnp.min(jnp.where(masked == m2, lane, E))
        sel2 = lane == i2
        dbuf[i] = jnp.where(sel1, m1, 0.0) + jnp.where(sel2, m2, 0.0)
        craw = jnp.where(sel1 | sel2, probs, 0.0)
        cbuf[i] = craw / jnp.maximum(jnp.sum(craw), 1e-8)
        return carry

    lax.fori_loop(0, tok_w, step, 0)
    pltpu.sync_copy(dbuf, disp_hbm.at[pl.ds(base, tok_w)])
    pltpu.sync_copy(cbuf, comb_hbm.at[pl.ds(base, tok_w)])


def _routing_sc(logits2d, tcoef):
    tok_w = (B * NB) // NWORKERS
    mesh = plsc.VectorSubcoreMesh(core_axis_name="c", subcore_axis_name="s")
    fn = pl.kernel(
        _route_body,
        out_type=(jax.ShapeDtypeStruct((B * NB, E), jnp.float32),
                  jax.ShapeDtypeStruct((B * NB, E), jnp.float32)),
        mesh=mesh,
        compiler_params=pltpu.CompilerParams(needs_layout_passes=False),
        scratch_types=[
            pltpu.VMEM((tok_w, E), jnp.float32),
            pltpu.VMEM((E,), jnp.float32),
            pltpu.VMEM((tok_w, E), jnp.float32),
            pltpu.VMEM((tok_w, E), jnp.float32),
        ],
    )
    return fn(logits2d, tcoef)


def kernel(tokens_B, output_A, output_C, t, Wq, bq, Wk, bk, Wv, bv, Wo, bo,
           ln_g, ln_b, Wt1, bt1, Wt2, bt2, Wg1, bg1, Wg2, bg2, expert_bias):
    f32 = jnp.float32
    ctx_AC = jnp.concatenate([output_A, output_C], axis=1)
    t_f = t.astype(f32).reshape(1, B)
    half = D // 2
    freq = jnp.exp(-math.log(10000.0) * jnp.arange(half, dtype=f32)
                   / (half - 1)).reshape(1, half)
    row = lambda a: a.reshape(1, -1)

    whole = lambda r, c: pl.BlockSpec((r, c), lambda b: (0, 0))
    k_out, v_out, tg, pen = pl.pallas_call(
        _ctx_body,
        grid=(B,),
        in_specs=[
            pl.BlockSpec((1, NK, D), lambda b: (b, 0, 0)),
            whole(D, D), whole(1, D), whole(D, D), whole(1, D),
            pl.BlockSpec(memory_space=pltpu.SMEM),
            whole(1, half),
            whole(D, 2 * D), whole(1, 2 * D), whole(2 * D, D), whole(1, D),
            whole(1, E),
        ],
        out_specs=[
            pl.BlockSpec((1, NK, D), lambda b: (b, 0, 0)),
            pl.BlockSpec((1, NK, D), lambda b: (b, 0, 0)),
            pl.BlockSpec((1, 1, D), lambda b: (b, 0, 0)),
            pl.BlockSpec((1, 1), lambda b: (0, 0)),
        ],
        out_shape=[
            jax.ShapeDtypeStruct((B, NK, D), f32),
            jax.ShapeDtypeStruct((B, NK, D), f32),
            jax.ShapeDtypeStruct((B, 1, D), f32),
            jax.ShapeDtypeStruct((1, 1), f32),
        ],
    )(ctx_AC, Wk, row(bk), Wv, row(bv), t_f, freq,
      Wt1, row(bt1), Wt2, row(bt2), row(expert_bias))

    whole2 = lambda r, c: pl.BlockSpec((r, c), lambda b, q: (0, 0))
    logits = pl.pallas_call(
        _main_body,
        grid=(B, NQ),
        in_specs=[
            pl.BlockSpec((1, BLKQ, D), lambda b, q: (b, q, 0)),
            pl.BlockSpec((1, NK, D), lambda b, q: (b, 0, 0)),
            pl.BlockSpec((1, NK, D), lambda b, q: (b, 0, 0)),
            whole2(D, D), whole2(1, D), whole2(D, D), whole2(1, D),
            whole2(1, D), whole2(1, D),
            whole2(3 * D, D),
            pl.BlockSpec((1, 1, D), lambda b, q: (b, 0, 0)),
            whole2(D, E), whole2(1, E),
        ],
        out_specs=pl.BlockSpec((1, BLKQ, E), lambda b, q: (b, q, 0)),
        out_shape=jax.ShapeDtypeStruct((B, NB, E), f32),
    )(tokens_B, k_out, v_out, Wq, row(bq), Wo, row(bo),
      row(ln_g), row(ln_b), Wg1, tg, Wg2,
      row(bg2) + row(expert_bias))

    tn = t.astype(f32) / TMAX
    tcoef = jnp.repeat(tn, NWORKERS // B)[:, None] * jnp.ones((1, E), f32)
    disp, comb = _routing_sc(logits.reshape(B * NB, E), tcoef)
    dispatch = disp.reshape(B, NB, E)
    combine = comb.reshape(B, NB, E)
    return dispatch, combine, pen.reshape(())
